# baseline (speedup 1.0000x reference)
import functools

import jax
import jax.numpy as jnp
from jax import lax
from jax.experimental import pallas as pl
from jax.experimental.pallas import tpu as pltpu
from jax.experimental.pallas import tpu_sc as plsc

N = 10000
MAX_NB = 32
D = 128
EPS = 1e-6

NC = 2
NS = 16
NW = NC * NS
NP = 10240
BPW = NP // NW
CH = 4
EC = CH * MAX_NB
NCHUNK = BPW // CH
TOTAL_CHUNKS = NP // CH
K0 = 128
K1 = 32
PADC = TOTAL_CHUNKS + K0

BLK = 512


def _selu(x):
    return 1.0507009873554805 * jnp.where(
        x > 0, x, 1.6732632423543772 * (jnp.exp(x) - 1.0))


def _lane_masks():
    lane = lax.broadcasted_iota(jnp.int32, (1, D), 1)
    e0 = (lane == 0).astype(jnp.float32)
    m0 = (lane > 0).astype(jnp.float32)
    return e0, m0


def _g_transform(msg, e0, m0):
    tcol = jnp.sum(msg * e0, axis=1, keepdims=True)
    xk = (msg * m0) / tcol
    n2 = jnp.clip(jnp.sum(xk * xk, axis=1, keepdims=True), 0.0, 0.9)
    lf = lax.rsqrt(1.0 - n2)
    return xk * lf + e0 * lf


def _tc_pre_body(nr_ref, lw_ref, lb_ref, wl_ref, out_ref):
    e0, m0 = _lane_masks()
    x = jnp.dot(nr_ref[...], lw_ref[...], preferred_element_type=jnp.float32)
    x = _selu(x + lb_ref[0:1, :])
    v = x * m0
    ldv = jnp.sum(v * v, axis=1, keepdims=True)
    nd = jnp.sqrt(jnp.maximum(ldv, EPS))
    t = jnp.minimum(nd, 1.0)
    et = jnp.exp(t)
    emt = 1.0 / et
    cosh_t = 0.5 * (et + emt)
    sinh_t = 0.5 * (et - emt)
    h = cosh_t * e0 + (sinh_t / nd) * v
    msg = jnp.dot(h, wl_ref[...], preferred_element_type=jnp.float32)
    out_ref[...] = _g_transform(msg, e0, m0)


def _tc_post_body(agg_ref, wl_ref, out_ref, *, with_msg):
    e0, m0 = _lane_masks()
    a = agg_ref[...]
    den = jnp.maximum(jnp.sum(a * e0, axis=1, keepdims=True), EPS)
    k = (a * m0) / den
    n2k = jnp.minimum(jnp.sum(k * k, axis=1, keepdims=True), 0.9)
    sq = jnp.sqrt(1.0 - n2k)
    p = _selu((k / sq) / (1.0 / sq + 1.0))
    n2p = jnp.sum(p * p, axis=1, keepdims=True)
    sp2 = (2.0 * p) / jnp.maximum(1.0 - n2p, EPS)
    t2 = jnp.sqrt(1.0 + jnp.sum(sp2 * sp2, axis=1, keepdims=True))
    h = t2 * e0 + sp2
    if with_msg:
        msg = jnp.dot(h, wl_ref[...], preferred_element_type=jnp.float32)
        out_ref[...] = _g_transform(msg, e0, m0)
    else:
        out_ref[...] = h


def _row_spec():
    return pl.BlockSpec((BLK, D), lambda i: (i, 0))


def _full_spec():
    return pl.BlockSpec((D, D), lambda i: (0, 0))


def _tc_pre(nr, lin_W, lb, Wl):
    return pl.pallas_call(
        _tc_pre_body,
        grid=(NP // BLK,),
        in_specs=[_row_spec(), _full_spec(),
                  pl.BlockSpec((8, D), lambda i: (0, 0)), _full_spec()],
        out_specs=_row_spec(),
        out_shape=jax.ShapeDtypeStruct((NP, D), jnp.float32),
    )(nr, lin_W, lb, Wl)


def _tc_post(agg, Wl, with_msg):
    return pl.pallas_call(
        functools.partial(_tc_post_body, with_msg=with_msg),
        grid=(NP // BLK,),
        in_specs=[_row_spec(), _full_spec()],
        out_specs=_row_spec(),
        out_shape=jax.ShapeDtypeStruct((NP, D), jnp.float32),
    )(agg, Wl)


def _bcast_lane(v, lane):
    idx = jnp.full((16, 1), lane, jnp.int32)
    dn = lax.GatherDimensionNumbers(
        offset_dims=(), collapsed_slice_dims=(0,), start_index_map=(0,))
    return lax.gather(v, idx, dn, slice_sizes=(1,),
                      mode=lax.GatherScatterMode.PROMISE_IN_BOUNDS)


def _sc_agg_body(table_hbm, adj_hbm, w_hbm, out_hbm,
                 idx_v, rows0, rows1, wb0, wb1, oc0, oc1,
                 semr0, semr1, semw0, semw1, semo0, semo1):
    cid = lax.axis_index("c")
    sid = lax.axis_index("s")
    base = jnp.where(cid == 0, sid * K0, NS * K0 + sid * K1)
    nchunks = jnp.where(cid == 0, K0, K1)
    pltpu.sync_copy(adj_hbm.at[pl.ds(base, K0)], idx_v)

    def start(g, rows, wb, semr, semw):
        for q in range(EC // 16):
            iv = idx_v[g, pl.ds(q * 16, 16)]
            for l in range(16):
                pltpu.async_copy(table_hbm.at[iv[l]],
                                 rows.at[q * 16 + l], semr)
        pltpu.async_copy(w_hbm.at[base + g], wb, semw)

    def wait(g, rows, wb, semr, semw):
        pltpu.make_async_copy(table_hbm.at[pl.ds(0, EC)], rows, semr).wait()
        pltpu.make_async_copy(w_hbm.at[base + g], wb, semw).wait()

    def compute(rows, wb, oc):
        wv = [wb[pl.ds(q * 16, 16)] for q in range(8)]
        for d in range(CH):
            accs = [jnp.zeros((16,), jnp.float32) for _ in range(8)]
            for n in range(MAX_NB):
                e = d * MAX_NB + n
                wsp = _bcast_lane(wv[e // 16], e % 16)
                for c in range(8):
                    accs[c] = accs[c] + wsp * rows[e, pl.ds(c * 16, 16)]
            for c in range(8):
                oc[d, pl.ds(c * 16, 16)] = accs[c]

    def out_slice(g):
        return out_hbm.at[pl.ds((base + g) * CH, CH)]

    start(0, rows0, wb0, semr0, semw0)

    @pl.loop(0, nchunks, step=2)
    def _chunks(gg):
        start(gg + 1, rows1, wb1, semr1, semw1)
        wait(gg, rows0, wb0, semr0, semw0)

        @pl.when(gg > 0)
        def _():
            pltpu.make_async_copy(oc0, out_slice(gg), semo0).wait()
        compute(rows0, wb0, oc0)
        pltpu.async_copy(oc0, out_slice(gg), semo0)

        @pl.when(gg + 2 < nchunks)
        def _():
            start(gg + 2, rows0, wb0, semr0, semw0)

        wait(gg + 1, rows1, wb1, semr1, semw1)

        @pl.when(gg > 0)
        def _():
            pltpu.make_async_copy(oc1, out_slice(gg + 1), semo1).wait()
        compute(rows1, wb1, oc1)
        pltpu.async_copy(oc1, out_slice(gg + 1), semo1)

    pltpu.make_async_copy(oc0, out_slice(0), semo0).wait()
    pltpu.make_async_copy(oc1, out_slice(0), semo1).wait()


@functools.cache
def _sc_agg_kernel():
    return pl.kernel(
        _sc_agg_body,
        out_type=jax.ShapeDtypeStruct((NP, D), jnp.float32),
        mesh=plsc.VectorSubcoreMesh(
            core_axis_name="c", subcore_axis_name="s",
            num_cores=NC, num_subcores=NS),
        compiler_params=pltpu.CompilerParams(needs_layout_passes=False),
        scratch_types=[
            pltpu.VMEM((K0, EC), jnp.int32),
            pltpu.VMEM((EC, D), jnp.float32),
            pltpu.VMEM((EC, D), jnp.float32),
            pltpu.VMEM((EC,), jnp.float32),
            pltpu.VMEM((EC,), jnp.float32),
            pltpu.VMEM((CH, D), jnp.float32),
            pltpu.VMEM((CH, D), jnp.float32),
            pltpu.SemaphoreType.DMA,
            pltpu.SemaphoreType.DMA,
            pltpu.SemaphoreType.DMA,
            pltpu.SemaphoreType.DMA,
            pltpu.SemaphoreType.DMA,
            pltpu.SemaphoreType.DMA,
        ],
    )


def _sc_agg(G, adj, w):
    return _sc_agg_kernel()(G, adj, w)


def kernel(node_repr, adj_list, weight, lin_W, lin_b, M):
    nr = jnp.pad(node_repr, ((0, NP - N), (0, 0)))
    adj = jnp.pad(adj_list, ((0, NP - N), (0, 0))).reshape(TOTAL_CHUNKS, EC)
    adj = jnp.pad(adj, ((0, PADC - TOTAL_CHUNKS), (0, 0)))
    w = jnp.pad(weight, ((0, NP - N), (0, 0))).reshape(TOTAL_CHUNKS, EC)
    w = jnp.pad(w, ((0, PADC - TOTAL_CHUNKS), (0, 0)))
    lb = jnp.broadcast_to(lin_b.reshape(1, D), (8, D))
    top = jnp.zeros((1, D), jnp.float32).at[0, 0].set(1.0)
    rest = jnp.concatenate([jnp.zeros((D - 1, 1), jnp.float32), M], axis=1)
    Wl = jnp.concatenate([top, rest], axis=0)

    G1 = _tc_pre(nr, lin_W, lb, Wl)
    agg1 = _sc_agg(G1, adj, w)
    G2 = _tc_post(agg1, Wl, with_msg=True)
    agg2 = _sc_agg(G2, adj, w)
    h = _tc_post(agg2, Wl, with_msg=False)
    return h[:N]


# speedup vs baseline: 1.1047x; 1.1047x over previous
import functools

import jax
import jax.numpy as jnp
from jax import lax
from jax.experimental import pallas as pl
from jax.experimental.pallas import tpu as pltpu
from jax.experimental.pallas import tpu_sc as plsc

N = 10000
MAX_NB = 32
D = 128
EPS = 1e-6

NC = 2
NS = 16
NW = NC * NS
NP = 10240
BPW = NP // NW
CH = 4
EC = CH * MAX_NB
NCHUNK = BPW // CH
TOTAL_CHUNKS = NP // CH
K0 = 128
K1 = 32
DROWS = 64
PADC = TOTAL_CHUNKS + K0

BLK = 512


def _selu(x):
    return 1.0507009873554805 * jnp.where(
        x > 0, x, 1.6732632423543772 * (jnp.exp(x) - 1.0))


def _lane_masks():
    lane = lax.broadcasted_iota(jnp.int32, (1, D), 1)
    e0 = (lane == 0).astype(jnp.float32)
    m0 = (lane > 0).astype(jnp.float32)
    return e0, m0


def _g_transform(msg, e0, m0):
    tcol = jnp.sum(msg * e0, axis=1, keepdims=True)
    xk = (msg * m0) / tcol
    n2 = jnp.clip(jnp.sum(xk * xk, axis=1, keepdims=True), 0.0, 0.9)
    lf = lax.rsqrt(1.0 - n2)
    return xk * lf + e0 * lf


def _tc_pre_body(nr_ref, lw_ref, lb_ref, wl_ref, out_ref):
    e0, m0 = _lane_masks()
    x = jnp.dot(nr_ref[...], lw_ref[...], preferred_element_type=jnp.float32)
    x = _selu(x + lb_ref[0:1, :])
    v = x * m0
    ldv = jnp.sum(v * v, axis=1, keepdims=True)
    nd = jnp.sqrt(jnp.maximum(ldv, EPS))
    t = jnp.minimum(nd, 1.0)
    et = jnp.exp(t)
    emt = 1.0 / et
    cosh_t = 0.5 * (et + emt)
    sinh_t = 0.5 * (et - emt)
    h = cosh_t * e0 + (sinh_t / nd) * v
    msg = jnp.dot(h, wl_ref[...], preferred_element_type=jnp.float32)
    out_ref[...] = _g_transform(msg, e0, m0)


def _tc_post_body(agg_ref, wl_ref, out_ref, *, with_msg):
    e0, m0 = _lane_masks()
    a = agg_ref[...]
    den = jnp.maximum(jnp.sum(a * e0, axis=1, keepdims=True), EPS)
    k = (a * m0) / den
    n2k = jnp.minimum(jnp.sum(k * k, axis=1, keepdims=True), 0.9)
    sq = jnp.sqrt(1.0 - n2k)
    p = _selu((k / sq) / (1.0 / sq + 1.0))
    n2p = jnp.sum(p * p, axis=1, keepdims=True)
    sp2 = (2.0 * p) / jnp.maximum(1.0 - n2p, EPS)
    t2 = jnp.sqrt(1.0 + jnp.sum(sp2 * sp2, axis=1, keepdims=True))
    h = t2 * e0 + sp2
    if with_msg:
        msg = jnp.dot(h, wl_ref[...], preferred_element_type=jnp.float32)
        out_ref[...] = _g_transform(msg, e0, m0)
    else:
        out_ref[...] = h


def _row_spec():
    return pl.BlockSpec((BLK, D), lambda i: (i, 0))


def _full_spec():
    return pl.BlockSpec((D, D), lambda i: (0, 0))


def _tc_pre(nr, lin_W, lb, Wl):
    return pl.pallas_call(
        _tc_pre_body,
        grid=(NP // BLK,),
        in_specs=[_row_spec(), _full_spec(),
                  pl.BlockSpec((8, D), lambda i: (0, 0)), _full_spec()],
        out_specs=_row_spec(),
        out_shape=jax.ShapeDtypeStruct((NP, D), jnp.float32),
    )(nr, lin_W, lb, Wl)


def _tc_post(agg, Wl, with_msg):
    return pl.pallas_call(
        functools.partial(_tc_post_body, with_msg=with_msg),
        grid=(NP // BLK,),
        in_specs=[_row_spec(), _full_spec()],
        out_specs=_row_spec(),
        out_shape=jax.ShapeDtypeStruct((NP, D), jnp.float32),
    )(agg, Wl)


def _bcast_lane(v, lane):
    idx = jnp.full((16, 1), lane, jnp.int32)
    dn = lax.GatherDimensionNumbers(
        offset_dims=(), collapsed_slice_dims=(0,), start_index_map=(0,))
    return lax.gather(v, idx, dn, slice_sizes=(1,),
                      mode=lax.GatherScatterMode.PROMISE_IN_BOUNDS)


def _sc_agg_body(table_hbm, adj_hbm, w_hbm, out_hbm,
                 idx_v, rows0, rows1, wb0, wb1, oc0, oc1,
                 semr0, semr1, semw0, semw1, semo0, semo1, semd0, semd1):
    cid = lax.axis_index("c")
    sid = lax.axis_index("s")
    base = jnp.where(cid == 0, sid * K0, NS * K0 + sid * K1)
    nchunks = jnp.where(cid == 0, K0, K1)
    pltpu.sync_copy(adj_hbm.at[pl.ds(base, K0)], idx_v)

    SROWS = EC - DROWS

    def start(g, rows, wb, semr, semw, semd):
        pltpu.async_copy(table_hbm.at[idx_v.at[g, pl.ds(0, SROWS)]],
                         rows.at[pl.ds(0, SROWS)], semr)
        for q in range(DROWS // 16):
            iv = idx_v[g, pl.ds(SROWS + q * 16, 16)]
            for l in range(16):
                pltpu.async_copy(table_hbm.at[iv[l]],
                                 rows.at[SROWS + q * 16 + l], semd)
        pltpu.async_copy(w_hbm.at[base + g], wb, semw)

    def wait(g, rows, wb, semr, semw, semd):
        pltpu.make_async_copy(table_hbm.at[idx_v.at[g, pl.ds(0, SROWS)]],
                              rows.at[pl.ds(0, SROWS)], semr).wait()
        pltpu.make_async_copy(table_hbm.at[pl.ds(0, DROWS)],
                              rows.at[pl.ds(SROWS, DROWS)], semd).wait()
        pltpu.make_async_copy(w_hbm.at[base + g], wb, semw).wait()

    def compute(rows, wb, oc):
        wv = [wb[pl.ds(q * 16, 16)] for q in range(8)]
        for d in range(CH):
            accs = [jnp.zeros((16,), jnp.float32) for _ in range(8)]
            for n in range(MAX_NB):
                e = d * MAX_NB + n
                wsp = _bcast_lane(wv[e // 16], e % 16)
                for c in range(8):
                    accs[c] = accs[c] + wsp * rows[e, pl.ds(c * 16, 16)]
            for c in range(8):
                oc[d, pl.ds(c * 16, 16)] = accs[c]

    def out_slice(g):
        return out_hbm.at[pl.ds((base + g) * CH, CH)]

    start(0, rows0, wb0, semr0, semw0, semd0)

    @pl.loop(0, nchunks, step=2)
    def _chunks(gg):
        start(gg + 1, rows1, wb1, semr1, semw1, semd1)
        wait(gg, rows0, wb0, semr0, semw0, semd0)

        @pl.when(gg > 0)
        def _():
            pltpu.make_async_copy(oc0, out_slice(gg), semo0).wait()
        compute(rows0, wb0, oc0)
        pltpu.async_copy(oc0, out_slice(gg), semo0)

        @pl.when(gg + 2 < nchunks)
        def _():
            start(gg + 2, rows0, wb0, semr0, semw0, semd0)

        wait(gg + 1, rows1, wb1, semr1, semw1, semd1)

        @pl.when(gg > 0)
        def _():
            pltpu.make_async_copy(oc1, out_slice(gg + 1), semo1).wait()
        compute(rows1, wb1, oc1)
        pltpu.async_copy(oc1, out_slice(gg + 1), semo1)

    pltpu.make_async_copy(oc0, out_slice(0), semo0).wait()
    pltpu.make_async_copy(oc1, out_slice(0), semo1).wait()
    return


@functools.cache
def _sc_agg_kernel():
    return pl.kernel(
        _sc_agg_body,
        out_type=jax.ShapeDtypeStruct((NP, D), jnp.float32),
        mesh=plsc.VectorSubcoreMesh(
            core_axis_name="c", subcore_axis_name="s",
            num_cores=NC, num_subcores=NS),
        compiler_params=pltpu.CompilerParams(needs_layout_passes=False),
        scratch_types=[
            pltpu.VMEM((K0, EC), jnp.int32),
            pltpu.VMEM((EC, D), jnp.float32),
            pltpu.VMEM((EC, D), jnp.float32),
            pltpu.VMEM((EC,), jnp.float32),
            pltpu.VMEM((EC,), jnp.float32),
            pltpu.VMEM((CH, D), jnp.float32),
            pltpu.VMEM((CH, D), jnp.float32),
            pltpu.SemaphoreType.DMA,
            pltpu.SemaphoreType.DMA,
            pltpu.SemaphoreType.DMA,
            pltpu.SemaphoreType.DMA,
            pltpu.SemaphoreType.DMA,
            pltpu.SemaphoreType.DMA,
            pltpu.SemaphoreType.DMA,
            pltpu.SemaphoreType.DMA,
        ],
    )


def _sc_agg(G, adj, w):
    return _sc_agg_kernel()(G, adj, w)


def kernel(node_repr, adj_list, weight, lin_W, lin_b, M):
    nr = jnp.pad(node_repr, ((0, NP - N), (0, 0)))
    adj = jnp.pad(adj_list, ((0, NP - N), (0, 0))).reshape(TOTAL_CHUNKS, EC)
    adj = jnp.pad(adj, ((0, PADC - TOTAL_CHUNKS), (0, 0)))
    w = jnp.pad(weight, ((0, NP - N), (0, 0))).reshape(TOTAL_CHUNKS, EC)
    w = jnp.pad(w, ((0, PADC - TOTAL_CHUNKS), (0, 0)))
    lb = jnp.broadcast_to(lin_b.reshape(1, D), (8, D))
    top = jnp.zeros((1, D), jnp.float32).at[0, 0].set(1.0)
    rest = jnp.concatenate([jnp.zeros((D - 1, 1), jnp.float32), M], axis=1)
    Wl = jnp.concatenate([top, rest], axis=0)

    G1 = _tc_pre(nr, lin_W, lb, Wl)
    agg1 = _sc_agg(G1, adj, w)
    G2 = _tc_post(agg1, Wl, with_msg=True)
    agg2 = _sc_agg(G2, adj, w)
    h = _tc_post(agg2, Wl, with_msg=False)
    return h[:N]


# speedup vs baseline: 1.1212x; 1.0150x over previous
import functools

import jax
import jax.numpy as jnp
from jax import lax
from jax.experimental import pallas as pl
from jax.experimental.pallas import tpu as pltpu
from jax.experimental.pallas import tpu_sc as plsc

N = 10000
MAX_NB = 32
D = 128
EPS = 1e-6

NC = 2
NS = 16
NW = NC * NS
NP = 10240
BPW = NP // NW
CH = 4
EC = CH * MAX_NB
NCHUNK = BPW // CH
TOTAL_CHUNKS = NP // CH
K0 = 128
K1 = 32
PADC = TOTAL_CHUNKS + K0

BLK = 512


def _selu(x):
    return 1.0507009873554805 * jnp.where(
        x > 0, x, 1.6732632423543772 * (jnp.exp(x) - 1.0))


def _lane_masks():
    lane = lax.broadcasted_iota(jnp.int32, (1, D), 1)
    e0 = (lane == 0).astype(jnp.float32)
    m0 = (lane > 0).astype(jnp.float32)
    return e0, m0


def _g_transform(msg, e0, m0):
    tcol = jnp.sum(msg * e0, axis=1, keepdims=True)
    xk = (msg * m0) / tcol
    n2 = jnp.clip(jnp.sum(xk * xk, axis=1, keepdims=True), 0.0, 0.9)
    lf = lax.rsqrt(1.0 - n2)
    return xk * lf + e0 * lf


def _tc_pre_body(nr_ref, lw_ref, lb_ref, wl_ref, out_ref):
    e0, m0 = _lane_masks()
    x = jnp.dot(nr_ref[...], lw_ref[...], preferred_element_type=jnp.float32)
    x = _selu(x + lb_ref[0:1, :])
    v = x * m0
    ldv = jnp.sum(v * v, axis=1, keepdims=True)
    nd = jnp.sqrt(jnp.maximum(ldv, EPS))
    t = jnp.minimum(nd, 1.0)
    et = jnp.exp(t)
    emt = 1.0 / et
    cosh_t = 0.5 * (et + emt)
    sinh_t = 0.5 * (et - emt)
    h = cosh_t * e0 + (sinh_t / nd) * v
    msg = jnp.dot(h, wl_ref[...], preferred_element_type=jnp.float32)
    out_ref[...] = _g_transform(msg, e0, m0)


def _tc_post_body(agg_ref, wl_ref, out_ref, *, with_msg):
    e0, m0 = _lane_masks()
    a = agg_ref[...]
    den = jnp.maximum(jnp.sum(a * e0, axis=1, keepdims=True), EPS)
    k = (a * m0) / den
    n2k = jnp.minimum(jnp.sum(k * k, axis=1, keepdims=True), 0.9)
    sq = jnp.sqrt(1.0 - n2k)
    p = _selu((k / sq) / (1.0 / sq + 1.0))
    n2p = jnp.sum(p * p, axis=1, keepdims=True)
    sp2 = (2.0 * p) / jnp.maximum(1.0 - n2p, EPS)
    t2 = jnp.sqrt(1.0 + jnp.sum(sp2 * sp2, axis=1, keepdims=True))
    h = t2 * e0 + sp2
    if with_msg:
        msg = jnp.dot(h, wl_ref[...], preferred_element_type=jnp.float32)
        out_ref[...] = _g_transform(msg, e0, m0)
    else:
        out_ref[...] = h


def _row_spec():
    return pl.BlockSpec((BLK, D), lambda i: (i, 0))


def _full_spec():
    return pl.BlockSpec((D, D), lambda i: (0, 0))


def _tc_pre(nr, lin_W, lb, Wl):
    return pl.pallas_call(
        _tc_pre_body,
        grid=(NP // BLK,),
        in_specs=[_row_spec(), _full_spec(),
                  pl.BlockSpec((8, D), lambda i: (0, 0)), _full_spec()],
        out_specs=_row_spec(),
        out_shape=jax.ShapeDtypeStruct((NP, D), jnp.float32),
    )(nr, lin_W, lb, Wl)


def _tc_post(agg, Wl, with_msg):
    return pl.pallas_call(
        functools.partial(_tc_post_body, with_msg=with_msg),
        grid=(NP // BLK,),
        in_specs=[_row_spec(), _full_spec()],
        out_specs=_row_spec(),
        out_shape=jax.ShapeDtypeStruct((NP, D), jnp.float32),
    )(agg, Wl)


def _bcast_lane(v, lane):
    idx = jnp.full((16, 1), lane, jnp.int32)
    dn = lax.GatherDimensionNumbers(
        offset_dims=(), collapsed_slice_dims=(0,), start_index_map=(0,))
    return lax.gather(v, idx, dn, slice_sizes=(1,),
                      mode=lax.GatherScatterMode.PROMISE_IN_BOUNDS)


def _sc_agg_body(table_hbm, adj_hbm, w_hbm, out_hbm,
                 idx_v, rows0, rows1, wb0, wb1, oc0, oc1,
                 semr0, semr1, semw0, semw1, semo0, semo1):
    cid = lax.axis_index("c")
    sid = lax.axis_index("s")
    base = jnp.where(cid == 0, sid * K0, NS * K0 + sid * K1)
    nchunks = jnp.where(cid == 0, K0, K1)
    pltpu.sync_copy(adj_hbm.at[pl.ds(base, K0)], idx_v)

    def start(g, rows, wb, semr, semw):
        pltpu.async_copy(table_hbm.at[idx_v.at[g]], rows, semr)
        pltpu.async_copy(w_hbm.at[base + g], wb, semw)

    def wait(g, rows, wb, semr, semw):
        pltpu.make_async_copy(table_hbm.at[idx_v.at[g]], rows, semr).wait()
        pltpu.make_async_copy(w_hbm.at[base + g], wb, semw).wait()

    def compute(rows, wb, oc):
        wv = [wb[pl.ds(q * 16, 16)] for q in range(8)]
        for d in range(CH):
            accs = [jnp.zeros((16,), jnp.float32) for _ in range(8)]
            for n in range(MAX_NB):
                e = d * MAX_NB + n
                wsp = _bcast_lane(wv[e // 16], e % 16)
                for c in range(8):
                    accs[c] = accs[c] + wsp * rows[e, pl.ds(c * 16, 16)]
            for c in range(8):
                oc[d, pl.ds(c * 16, 16)] = accs[c]

    def out_slice(g):
        return out_hbm.at[pl.ds((base + g) * CH, CH)]

    start(0, rows0, wb0, semr0, semw0)

    @pl.loop(0, nchunks, step=2)
    def _chunks(gg):
        start(gg + 1, rows1, wb1, semr1, semw1)
        wait(gg, rows0, wb0, semr0, semw0)

        @pl.when(gg > 0)
        def _():
            pltpu.make_async_copy(oc0, out_slice(gg), semo0).wait()
        compute(rows0, wb0, oc0)
        pltpu.async_copy(oc0, out_slice(gg), semo0)

        @pl.when(gg + 2 < nchunks)
        def _():
            start(gg + 2, rows0, wb0, semr0, semw0)

        wait(gg + 1, rows1, wb1, semr1, semw1)

        @pl.when(gg > 0)
        def _():
            pltpu.make_async_copy(oc1, out_slice(gg + 1), semo1).wait()
        compute(rows1, wb1, oc1)
        pltpu.async_copy(oc1, out_slice(gg + 1), semo1)

    pltpu.make_async_copy(oc0, out_slice(0), semo0).wait()
    pltpu.make_async_copy(oc1, out_slice(0), semo1).wait()


@functools.cache
def _sc_agg_kernel():
    return pl.kernel(
        _sc_agg_body,
        out_type=jax.ShapeDtypeStruct((NP, D), jnp.float32),
        mesh=plsc.VectorSubcoreMesh(
            core_axis_name="c", subcore_axis_name="s",
            num_cores=NC, num_subcores=NS),
        compiler_params=pltpu.CompilerParams(needs_layout_passes=False),
        scratch_types=[
            pltpu.VMEM((K0, EC), jnp.int32),
            pltpu.VMEM((EC, D), jnp.float32),
            pltpu.VMEM((EC, D), jnp.float32),
            pltpu.VMEM((EC,), jnp.float32),
            pltpu.VMEM((EC,), jnp.float32),
            pltpu.VMEM((CH, D), jnp.float32),
            pltpu.VMEM((CH, D), jnp.float32),
            pltpu.SemaphoreType.DMA,
            pltpu.SemaphoreType.DMA,
            pltpu.SemaphoreType.DMA,
            pltpu.SemaphoreType.DMA,
            pltpu.SemaphoreType.DMA,
            pltpu.SemaphoreType.DMA,
        ],
    )


def _sc_agg(G, adj, w):
    return _sc_agg_kernel()(G, adj, w)


def kernel(node_repr, adj_list, weight, lin_W, lin_b, M):
    nr = jnp.pad(node_repr, ((0, NP - N), (0, 0)))
    adj = jnp.pad(adj_list, ((0, NP - N), (0, 0))).reshape(TOTAL_CHUNKS, EC)
    adj = jnp.pad(adj, ((0, PADC - TOTAL_CHUNKS), (0, 0)))
    w = jnp.pad(weight, ((0, NP - N), (0, 0))).reshape(TOTAL_CHUNKS, EC)
    w = jnp.pad(w, ((0, PADC - TOTAL_CHUNKS), (0, 0)))
    lb = jnp.broadcast_to(lin_b.reshape(1, D), (8, D))
    top = jnp.zeros((1, D), jnp.float32).at[0, 0].set(1.0)
    rest = jnp.concatenate([jnp.zeros((D - 1, 1), jnp.float32), M], axis=1)
    Wl = jnp.concatenate([top, rest], axis=0)

    G1 = _tc_pre(nr, lin_W, lb, Wl)
    agg1 = _sc_agg(G1, adj, w)
    G2 = _tc_post(agg1, Wl, with_msg=True)
    agg2 = _sc_agg(G2, adj, w)
    h = _tc_post(agg2, Wl, with_msg=False)
    return h[:N]
